# baseline (device time: 8531 ns/iter reference)
import jax
import jax.numpy as jnp
from jax import lax
from jax.experimental import pallas as pl
from jax.experimental.pallas import tpu as pltpu

N_CHUNKS = 2


def kernel(x):
    _, m, n_total = x.shape
    n_out = n_total // 2
    m_chunk = m // N_CHUNKS

    def body(x_ref, out_ref, send_buf, comm_ref, send_sems, recv_sems):
        my_x = lax.axis_index("x")
        my_y = lax.axis_index("y")
        my_z = lax.axis_index("z")
        partner = (my_x, 1 - my_y, my_z)

        partner_off = (1 - my_y) * n_out
        my_off = my_y * n_out

        send_buf[:, :] = x_ref[0, :, pl.ds(partner_off, n_out)]

        barrier = pltpu.get_barrier_semaphore()
        pl.semaphore_signal(
            barrier, inc=1, device_id=partner,
            device_id_type=pl.DeviceIdType.MESH,
        )
        pl.semaphore_wait(barrier, 1)

        rdmas = []
        for c in range(N_CHUNKS):
            rows = pl.ds(c * m_chunk, m_chunk)
            rdma = pltpu.make_async_remote_copy(
                src_ref=send_buf.at[rows],
                dst_ref=comm_ref.at[rows],
                send_sem=send_sems.at[c],
                recv_sem=recv_sems.at[c],
                device_id=partner,
                device_id_type=pl.DeviceIdType.MESH,
            )
            rdma.start()
            rdmas.append(rdma)

        for c in range(N_CHUNKS):
            rows = pl.ds(c * m_chunk, m_chunk)
            rdmas[c].wait_recv()
            out_ref[rows, :] = comm_ref[rows, :] + x_ref[0, rows, pl.ds(my_off, n_out)]
        for c in range(N_CHUNKS):
            rdmas[c].wait_send()

    return pl.pallas_call(
        body,
        out_shape=jax.ShapeDtypeStruct((m, n_out), x.dtype),
        in_specs=[pl.BlockSpec(memory_space=pltpu.VMEM)],
        out_specs=pl.BlockSpec(memory_space=pltpu.VMEM),
        scratch_shapes=[
            pltpu.VMEM((m, n_out), x.dtype),
            pltpu.VMEM((m, n_out), x.dtype),
            pltpu.SemaphoreType.DMA((N_CHUNKS,)),
            pltpu.SemaphoreType.DMA((N_CHUNKS,)),
        ],
        compiler_params=pltpu.CompilerParams(collective_id=0),
    )(x)


# device time: 8507 ns/iter; 1.0028x vs baseline; 1.0028x over previous
import jax
import jax.numpy as jnp
from jax import lax
from jax.experimental import pallas as pl
from jax.experimental.pallas import tpu as pltpu

N_CHUNKS = 2


def kernel(x):
    _, m, n_total = x.shape
    n_out = n_total // 2
    m_chunk = m // N_CHUNKS

    def body(x_ref, out_ref, comm_ref, send_sems, recv_sems):
        my_x = lax.axis_index("x")
        my_y = lax.axis_index("y")
        my_z = lax.axis_index("z")
        partner = (my_x, 1 - my_y, my_z)

        barrier = pltpu.get_barrier_semaphore()
        pl.semaphore_signal(
            barrier, inc=1, device_id=partner,
            device_id_type=pl.DeviceIdType.MESH,
        )
        pl.semaphore_wait(barrier, 1)

        partner_off = (1 - my_y) * n_out
        my_off = my_y * n_out

        rdmas = []
        for c in range(N_CHUNKS):
            rows = pl.ds(c * m_chunk, m_chunk)
            rdma = pltpu.make_async_remote_copy(
                src_ref=x_ref.at[0, rows, pl.ds(partner_off, n_out)],
                dst_ref=comm_ref.at[rows],
                send_sem=send_sems.at[c],
                recv_sem=recv_sems.at[c],
                device_id=partner,
                device_id_type=pl.DeviceIdType.MESH,
            )
            rdma.start()
            rdmas.append(rdma)

        out_ref[:, :] = x_ref[0, :, pl.ds(my_off, n_out)]

        for c in range(N_CHUNKS):
            rows = pl.ds(c * m_chunk, m_chunk)
            rdmas[c].wait_recv()
            out_ref[rows, :] += comm_ref[rows, :]
        for c in range(N_CHUNKS):
            rdmas[c].wait_send()

    return pl.pallas_call(
        body,
        out_shape=jax.ShapeDtypeStruct((m, n_out), x.dtype),
        in_specs=[pl.BlockSpec(memory_space=pltpu.VMEM)],
        out_specs=pl.BlockSpec(memory_space=pltpu.VMEM),
        scratch_shapes=[
            pltpu.VMEM((m, n_out), x.dtype),
            pltpu.SemaphoreType.DMA((N_CHUNKS,)),
            pltpu.SemaphoreType.DMA((N_CHUNKS,)),
        ],
        compiler_params=pltpu.CompilerParams(collective_id=0),
    )(x)
